# baseline (speedup 1.0000x reference)
import functools

import jax
import jax.numpy as jnp
from jax import lax
from jax.experimental import pallas as pl
from jax.experimental.pallas import tpu as pltpu
from jax.experimental.pallas import tpu_sc as plsc

D_MODEL = 64
_SCALE = 8.0
_NC, _NS, _L = 2, 16, 16
_NW = _NC * _NS
_CHUNK = 512


@functools.partial(jax.jit, static_argnums=(0,))
def _sc_lookup(B, xf, table):
    per_w = B // _NW
    n_chunks = per_w // _CHUNK
    mesh = plsc.VectorSubcoreMesh(core_axis_name="c", subcore_axis_name="s")

    @functools.partial(
        pl.kernel,
        mesh=mesh,
        out_type=jax.ShapeDtypeStruct((B, D_MODEL), jnp.float32),
        scratch_types=[
            pltpu.VMEM((_CHUNK,), jnp.int32),
            pltpu.VMEM((_CHUNK, D_MODEL), jnp.float32),
            pltpu.SemaphoreType.DMA,
        ],
        compiler_params=pltpu.CompilerParams(use_tc_tiling_on_sc=False),
    )
    def k(x_hbm, table_hbm, out_hbm, idx_v, rows_v, sem):
        wid = lax.axis_index("s") * _NC + lax.axis_index("c")
        base = wid * per_w

        def chunk_body(c, carry):
            off = base + c * _CHUNK
            pltpu.sync_copy(x_hbm.at[pl.ds(off, _CHUNK)], idx_v)
            pltpu.async_copy(table_hbm.at[idx_v], rows_v, sem).wait()

            def scale_body(i, carry2):
                for j in range(D_MODEL // _L):
                    s = (i, pl.ds(j * _L, _L))
                    rows_v[s] = rows_v[s] * _SCALE
                return carry2

            lax.fori_loop(0, _CHUNK, scale_body, 0, unroll=2)
            pltpu.sync_copy(rows_v, out_hbm.at[pl.ds(off, _CHUNK)])
            return carry

        lax.fori_loop(0, n_chunks, chunk_body, 0)

    return k(xf, table)


def kernel(x, table):
    lead_shape = x.shape
    xf = x.reshape(-1).astype(jnp.int32)
    out = _sc_lookup(xf.shape[0], xf, table)
    return out.reshape(*lead_shape, D_MODEL)


# speedup vs baseline: 1.0914x; 1.0914x over previous
import functools

import jax
import jax.numpy as jnp
from jax import lax
from jax.experimental import pallas as pl
from jax.experimental.pallas import tpu as pltpu
from jax.experimental.pallas import tpu_sc as plsc

D_MODEL = 64
_SCALE = 8.0
_NC, _NS, _L = 2, 16, 16
_NW = _NC * _NS
_C = 256
_NBUF = 4


@functools.partial(jax.jit, static_argnums=(0,))
def _sc_lookup(B, xf, table):
    per_w = B // _NW
    n_chunks = per_w // _C
    groups = n_chunks // _NBUF
    mesh = plsc.VectorSubcoreMesh(core_axis_name="c", subcore_axis_name="s")

    @functools.partial(
        pl.kernel,
        mesh=mesh,
        out_type=jax.ShapeDtypeStruct((B, D_MODEL), jnp.float32),
        scratch_types=(
            [pltpu.VMEM((per_w,), jnp.int32)]
            + [pltpu.VMEM((_C, D_MODEL), jnp.float32) for _ in range(_NBUF)]
            + [pltpu.SemaphoreType.DMA for _ in range(2 * _NBUF)]
        ),
        compiler_params=pltpu.CompilerParams(use_tc_tiling_on_sc=False),
    )
    def k(x_hbm, table_hbm, out_hbm, idx_v, *bufs_and_sems):
        rows = bufs_and_sems[:_NBUF]
        gsem = bufs_and_sems[_NBUF : 2 * _NBUF]
        ssem = bufs_and_sems[2 * _NBUF :]
        wid = lax.axis_index("s") * _NC + lax.axis_index("c")
        base = wid * per_w
        pltpu.sync_copy(x_hbm.at[pl.ds(base, per_w)], idx_v)

        def fire_g(c, b):
            pltpu.async_copy(
                table_hbm.at[idx_v.at[pl.ds(c * _C, _C)]], rows[b], gsem[b]
            )

        def wait_g(c, b):
            pltpu.make_async_copy(
                table_hbm.at[idx_v.at[pl.ds(c * _C, _C)]], rows[b], gsem[b]
            ).wait()

        def fire_s(c, b):
            pltpu.async_copy(rows[b], out_hbm.at[pl.ds(base + c * _C, _C)], ssem[b])

        def wait_s(c, b):
            pltpu.make_async_copy(
                rows[b], out_hbm.at[pl.ds(base + c * _C, _C)], ssem[b]
            ).wait()

        def scale(b):
            buf = rows[b]

            @plsc.parallel_loop(0, _C, unroll=4)
            def _(i):
                for j in range(D_MODEL // _L):
                    s = (i, pl.ds(j * _L, _L))
                    buf[s] = buf[s] * _SCALE

        def chunk_body(c, b, refire):
            wait_g(c, b)
            scale(b)
            fire_s(c, b)
            if refire:
                bp = (b - 1) % _NBUF
                wait_s(c - 1, bp)
                fire_g(c - 1 + _NBUF, bp)

        for b in range(_NBUF):
            fire_g(b, b)
        for b in range(_NBUF):
            chunk_body(b, b, refire=b >= 1)

        def group_body(g, carry):
            c0 = g * _NBUF
            for b in range(_NBUF):
                chunk_body(c0 + b, b, refire=True)
            return carry

        lax.fori_loop(1, groups - 1, group_body, 0)

        c0 = (groups - 1) * _NBUF
        for b in range(_NBUF):
            chunk_body(c0 + b, b, refire=b == 0)
        for b in range(_NBUF):
            wait_s(c0 + b, b)

    return k(xf, table)


def kernel(x, table):
    lead_shape = x.shape
    xf = x.reshape(-1).astype(jnp.int32)
    out = _sc_lookup(xf.shape[0], xf, table)
    return out.reshape(*lead_shape, D_MODEL)
